# baseline (speedup 1.0000x reference)
import functools

import jax
import jax.numpy as jnp
from jax import lax
from jax.experimental import pallas as pl
from jax.experimental.pallas import tpu as pltpu
from jax.experimental.pallas import tpu_sc as plsc

N = 10000
D = 128
NC = 2
NS = 16
NW = NC * NS
C = 128
NP = 10240
ROWS_PER_TILE = NP // NS
CNT_PER_TILE = NP * 8 // NS
RB = 1000
MP = 2560


def _enc_body(x_ref, w_ref, mi_ref, a_ref, h_ref):
    i = pl.program_id(0)
    x = x_ref[...]
    xp = jnp.where(x >= 0, x, a_ref[...] * x)
    h = lax.dot_general(xp, w_ref[...], (((1,), (1,)), ((), ())),
                        preferred_element_type=jnp.float32,
                        precision=lax.Precision.HIGHEST)
    row = lax.broadcasted_iota(jnp.int32, (RB, 1), 0) + i * RB
    hit = jnp.zeros((RB, 1), jnp.bool_)
    for j in range(MP // 128):
        blk = mi_ref[j:j + 1, :]
        hit = jnp.logical_or(hit, jnp.any(row == blk, axis=1, keepdims=True))
    h_ref[...] = jnp.where(hit, 0.0, h)


def _mlp_body(ag_ref, c1_ref, c2_ref, h_ref, e1_ref, e2_ref,
              w1_ref, b1_ref, w2_ref, b2_ref, out_ref):
    agg = ag_ref[0] + ag_ref[1] + h_ref[...]
    agg = agg + (e1_ref[4:5, :] + e2_ref[0:1, :])
    c1 = c1_ref[0] + c1_ref[1]
    c2 = c2_ref[0] + c2_ref[1]
    agg = agg + lax.dot_general(c1, e1_ref[...], (((1,), (0,)), ((), ())),
                                preferred_element_type=jnp.float32,
                                precision=lax.Precision.HIGHEST)
    agg = agg + lax.dot_general(c2, e2_ref[...], (((1,), (0,)), ((), ())),
                                preferred_element_type=jnp.float32,
                                precision=lax.Precision.HIGHEST)
    z = lax.dot_general(agg, w1_ref[...], (((1,), (1,)), ((), ())),
                        preferred_element_type=jnp.float32,
                        precision=lax.Precision.HIGHEST) + b1_ref[...]
    z = jnp.maximum(z, 0.0)
    out_ref[...] = lax.dot_general(z, w2_ref[...], (((1,), (1,)), ((), ())),
                                   preferred_element_type=jnp.float32,
                                   precision=lax.Precision.HIGHEST) + b2_ref[...]


CH0 = 88
CH1 = 70


def _make_sc_aggregate(ch0, ch1):
    mesh = plsc.VectorSubcoreMesh(core_axis_name="c", subcore_axis_name="s",
                                  num_cores=NC, num_subcores=NS)

    @functools.partial(
        pl.kernel,
        out_type=[
            jax.ShapeDtypeStruct((NC, NP, D), jnp.float32),
            jax.ShapeDtypeStruct((NC, NP * 8), jnp.float32),
            jax.ShapeDtypeStruct((NC, NP * 8), jnp.float32),
        ],
        mesh=mesh,
        scratch_types=[
            pltpu.VMEM((4, C), jnp.int32),
            pltpu.VMEM((C,), jnp.int32),
            pltpu.VMEM((C,), jnp.int32),
            pltpu.VMEM((C, D), jnp.float32),
            pltpu.VMEM((C,), jnp.float32),
            pltpu.VMEM_SHARED((NP, D), jnp.float32),
            pltpu.VMEM_SHARED((NP * 8,), jnp.float32),
            pltpu.VMEM_SHARED((NP * 8,), jnp.float32),
        ],
    )
    def sc_aggregate(epk_hbm, h_hbm, zrows_hbm, zflat_hbm, ones_hbm,
                     aggr_out, c1_out, c2_out,
                     edg_v, idx1_v, idx2_v, rows_v, ones_v,
                     aggr_sh, c1_sh, c2_sh):

        c = lax.axis_index("c")
        s = lax.axis_index("s")
        wid = c * NS + s
        pltpu.sync_copy(zrows_hbm, aggr_sh.at[pl.ds(s * ROWS_PER_TILE, ROWS_PER_TILE)])
        pltpu.sync_copy(zflat_hbm, c1_sh.at[pl.ds(s * CNT_PER_TILE, CNT_PER_TILE)])
        pltpu.sync_copy(zflat_hbm, c2_sh.at[pl.ds(s * CNT_PER_TILE, CNT_PER_TILE)])
        pltpu.sync_copy(ones_hbm, ones_v)
        plsc.subcore_barrier()

        nloc = jnp.where(c == 0, ch0, ch1)
        base = jnp.where(c == 0, s * ch0, NS * ch0 + s * ch1)

        @pl.loop(0, nloc)
        def _chunk(k):
            pltpu.sync_copy(epk_hbm.at[base + k], edg_v)
            pltpu.sync_copy(h_hbm.at[edg_v.at[0]], rows_v)
            for i in range(C // 16):
                sl = pl.ds(i * 16, 16)
                dsl = edg_v[1, sl]
                idx1_v[sl] = dsl * 8 + edg_v[2, sl]
                idx2_v[sl] = dsl * 8 + edg_v[3, sl]
            pltpu.sync_copy(rows_v, aggr_sh.at[edg_v.at[1]], add=True)
            pltpu.sync_copy(ones_v, c1_sh.at[idx1_v], add=True)
            pltpu.sync_copy(ones_v, c2_sh.at[idx2_v], add=True)

        plsc.subcore_barrier()
        rsl = pl.ds(s * ROWS_PER_TILE, ROWS_PER_TILE)
        csl = pl.ds(s * CNT_PER_TILE, CNT_PER_TILE)
        pltpu.sync_copy(aggr_sh.at[rsl], aggr_out.at[c, rsl])
        pltpu.sync_copy(c1_sh.at[csl], c1_out.at[c, csl])
        pltpu.sync_copy(c2_sh.at[csl], c2_out.at[c, csl])

    return sc_aggregate


def kernel(x, edge_index, edge_attr, mask_node_indices, prelu_a,
           W_enc, emb1, emb2, W1, b1, W2, b2):
    e = edge_index.shape[1]
    tch = NS * (CH0 + CH1)
    assert tch * C >= e
    epad = tch * C
    src = jnp.concatenate([edge_index[0].astype(jnp.int32),
                           jnp.zeros((epad - e,), jnp.int32)])
    dst = jnp.concatenate([edge_index[1].astype(jnp.int32),
                           jnp.full((epad - e,), N, jnp.int32)])
    ea0 = jnp.concatenate([edge_attr[:, 0].astype(jnp.int32),
                           jnp.zeros((epad - e,), jnp.int32)])
    ea1 = jnp.concatenate([edge_attr[:, 1].astype(jnp.int32),
                           jnp.zeros((epad - e,), jnp.int32)])
    epk = jnp.stack([src.reshape(-1, C), dst.reshape(-1, C),
                     ea0.reshape(-1, C), ea1.reshape(-1, C)], axis=1)
    mi = jnp.concatenate([mask_node_indices.astype(jnp.int32),
                          jnp.full((MP - mask_node_indices.shape[0],), N,
                                   jnp.int32)]).reshape(MP // 128, 128)
    a11 = jnp.reshape(prelu_a.astype(jnp.float32), (1, 1))
    e1p = jnp.pad(emb1, ((0, 8 - emb1.shape[0]), (0, 0)))
    e2p = jnp.pad(emb2, ((0, 8 - emb2.shape[0]), (0, 0)))
    zrows = jnp.zeros((ROWS_PER_TILE, D), jnp.float32)
    zflat = jnp.zeros((CNT_PER_TILE,), jnp.float32)
    ones128 = jnp.ones((C,), jnp.float32)

    h = pl.pallas_call(
        _enc_body,
        grid=(N // RB,),
        in_specs=[
            pl.BlockSpec((RB, D), lambda i: (i, 0)),
            pl.BlockSpec((D, D), lambda i: (0, 0)),
            pl.BlockSpec((MP // 128, 128), lambda i: (0, 0)),
            pl.BlockSpec((1, 1), lambda i: (0, 0)),
        ],
        out_specs=pl.BlockSpec((RB, D), lambda i: (i, 0)),
        out_shape=jax.ShapeDtypeStruct((N, D), jnp.float32),
    )(x, W_enc, mi, a11)

    aggr, c1f, c2f = _make_sc_aggregate(CH0, CH1)(epk, h, zrows, zflat, ones128)
    c1 = c1f.reshape(NC, NP, 8)
    c2 = c2f.reshape(NC, NP, 8)

    out = pl.pallas_call(
        _mlp_body,
        grid=(N // RB,),
        in_specs=[
            pl.BlockSpec((NC, RB, D), lambda i: (0, i, 0)),
            pl.BlockSpec((NC, RB, 8), lambda i: (0, i, 0)),
            pl.BlockSpec((NC, RB, 8), lambda i: (0, i, 0)),
            pl.BlockSpec((RB, D), lambda i: (i, 0)),
            pl.BlockSpec((8, D), lambda i: (0, 0)),
            pl.BlockSpec((8, D), lambda i: (0, 0)),
            pl.BlockSpec((2 * D, D), lambda i: (0, 0)),
            pl.BlockSpec((1, 2 * D), lambda i: (0, 0)),
            pl.BlockSpec((D, 2 * D), lambda i: (0, 0)),
            pl.BlockSpec((1, D), lambda i: (0, 0)),
        ],
        out_specs=pl.BlockSpec((RB, D), lambda i: (i, 0)),
        out_shape=jax.ShapeDtypeStruct((N, D), jnp.float32),
    )(aggr, c1, c2, h, e1p, e2p, W1, b1.reshape(1, 2 * D), W2,
      b2.reshape(1, D))
    return out


# speedup vs baseline: 1.0384x; 1.0384x over previous
import functools

import jax
import jax.numpy as jnp
from jax import lax
from jax.experimental import pallas as pl
from jax.experimental.pallas import tpu as pltpu
from jax.experimental.pallas import tpu_sc as plsc

N = 10000
D = 128
NC = 2
NS = 16
NW = NC * NS
C = 128
NP = 10240
ROWS_PER_TILE = NP // NS
CNT_PER_TILE = NP * 8 // NS
RB = 1000
MP = 2560


def _enc_body(x_ref, w_ref, mi_ref, a_ref, h_ref):
    i = pl.program_id(0)
    x = x_ref[...]
    xp = jnp.where(x >= 0, x, a_ref[...] * x)
    h = lax.dot_general(xp, w_ref[...], (((1,), (1,)), ((), ())),
                        preferred_element_type=jnp.float32,
                        precision=lax.Precision.HIGHEST)
    row = lax.broadcasted_iota(jnp.int32, (RB, 1), 0) + i * RB
    hit = jnp.zeros((RB, 1), jnp.bool_)
    for j in range(MP // 128):
        blk = mi_ref[j:j + 1, :]
        hit = jnp.logical_or(hit, jnp.any(row == blk, axis=1, keepdims=True))
    h_ref[...] = jnp.where(hit, 0.0, h)


def _mlp_body(ag_ref, c1_ref, c2_ref, h_ref, e1_ref, e2_ref,
              w1_ref, b1_ref, w2_ref, b2_ref, out_ref):
    agg = ag_ref[0] + ag_ref[1] + h_ref[...]
    agg = agg + (e1_ref[4:5, :] + e2_ref[0:1, :])
    c1 = c1_ref[0] + c1_ref[1]
    c2 = c2_ref[0] + c2_ref[1]
    agg = agg + lax.dot_general(c1, e1_ref[...], (((1,), (0,)), ((), ())),
                                preferred_element_type=jnp.float32,
                                precision=lax.Precision.HIGHEST)
    agg = agg + lax.dot_general(c2, e2_ref[...], (((1,), (0,)), ((), ())),
                                preferred_element_type=jnp.float32,
                                precision=lax.Precision.HIGHEST)
    z = lax.dot_general(agg, w1_ref[...], (((1,), (1,)), ((), ())),
                        preferred_element_type=jnp.float32,
                        precision=lax.Precision.HIGHEST) + b1_ref[...]
    z = jnp.maximum(z, 0.0)
    out_ref[...] = lax.dot_general(z, w2_ref[...], (((1,), (1,)), ((), ())),
                                   preferred_element_type=jnp.float32,
                                   precision=lax.Precision.HIGHEST) + b2_ref[...]


CH0 = 98
CH1 = 60


def _make_sc_aggregate(ch0, ch1):
    mesh = plsc.VectorSubcoreMesh(core_axis_name="c", subcore_axis_name="s",
                                  num_cores=NC, num_subcores=NS)

    @functools.partial(
        pl.kernel,
        out_type=[
            jax.ShapeDtypeStruct((NC, NP, D), jnp.float32),
            jax.ShapeDtypeStruct((NC, NP * 8), jnp.float32),
            jax.ShapeDtypeStruct((NC, NP * 8), jnp.float32),
        ],
        mesh=mesh,
        scratch_types=[
            pltpu.VMEM((4, C), jnp.int32),
            pltpu.VMEM((C,), jnp.int32),
            pltpu.VMEM((C,), jnp.int32),
            pltpu.VMEM((C, D), jnp.float32),
            pltpu.VMEM((C,), jnp.float32),
            pltpu.VMEM_SHARED((NP, D), jnp.float32),
            pltpu.VMEM_SHARED((NP * 8,), jnp.float32),
            pltpu.VMEM_SHARED((NP * 8,), jnp.float32),
        ],
    )
    def sc_aggregate(epk_hbm, h_hbm, zrows_hbm, zflat_hbm, ones_hbm,
                     aggr_out, c1_out, c2_out,
                     edg_v, idx1_v, idx2_v, rows_v, ones_v,
                     aggr_sh, c1_sh, c2_sh):

        c = lax.axis_index("c")
        s = lax.axis_index("s")
        wid = c * NS + s
        pltpu.sync_copy(zrows_hbm, aggr_sh.at[pl.ds(s * ROWS_PER_TILE, ROWS_PER_TILE)])
        pltpu.sync_copy(zflat_hbm, c1_sh.at[pl.ds(s * CNT_PER_TILE, CNT_PER_TILE)])
        pltpu.sync_copy(zflat_hbm, c2_sh.at[pl.ds(s * CNT_PER_TILE, CNT_PER_TILE)])
        pltpu.sync_copy(ones_hbm, ones_v)
        plsc.subcore_barrier()

        nloc = jnp.where(c == 0, ch0, ch1)
        base = jnp.where(c == 0, s * ch0, NS * ch0 + s * ch1)

        @pl.loop(0, nloc)
        def _chunk(k):
            pltpu.sync_copy(epk_hbm.at[base + k], edg_v)
            pltpu.sync_copy(h_hbm.at[edg_v.at[0]], rows_v)
            for i in range(C // 16):
                sl = pl.ds(i * 16, 16)
                dsl = edg_v[1, sl]
                idx1_v[sl] = dsl * 8 + edg_v[2, sl]
                idx2_v[sl] = dsl * 8 + edg_v[3, sl]
            pltpu.sync_copy(rows_v, aggr_sh.at[edg_v.at[1]], add=True)
            pltpu.sync_copy(ones_v, c1_sh.at[idx1_v], add=True)
            pltpu.sync_copy(ones_v, c2_sh.at[idx2_v], add=True)

        plsc.subcore_barrier()
        rsl = pl.ds(s * ROWS_PER_TILE, ROWS_PER_TILE)
        csl = pl.ds(s * CNT_PER_TILE, CNT_PER_TILE)
        pltpu.sync_copy(aggr_sh.at[rsl], aggr_out.at[c, rsl])
        pltpu.sync_copy(c1_sh.at[csl], c1_out.at[c, csl])
        pltpu.sync_copy(c2_sh.at[csl], c2_out.at[c, csl])

    return sc_aggregate


def kernel(x, edge_index, edge_attr, mask_node_indices, prelu_a,
           W_enc, emb1, emb2, W1, b1, W2, b2):
    e = edge_index.shape[1]
    tch = NS * (CH0 + CH1)
    assert tch * C >= e
    epad = tch * C
    src = jnp.concatenate([edge_index[0].astype(jnp.int32),
                           jnp.zeros((epad - e,), jnp.int32)])
    dst = jnp.concatenate([edge_index[1].astype(jnp.int32),
                           jnp.full((epad - e,), N, jnp.int32)])
    ea0 = jnp.concatenate([edge_attr[:, 0].astype(jnp.int32),
                           jnp.zeros((epad - e,), jnp.int32)])
    ea1 = jnp.concatenate([edge_attr[:, 1].astype(jnp.int32),
                           jnp.zeros((epad - e,), jnp.int32)])
    epk = jnp.stack([src.reshape(-1, C), dst.reshape(-1, C),
                     ea0.reshape(-1, C), ea1.reshape(-1, C)], axis=1)
    mi = jnp.concatenate([mask_node_indices.astype(jnp.int32),
                          jnp.full((MP - mask_node_indices.shape[0],), N,
                                   jnp.int32)]).reshape(MP // 128, 128)
    a11 = jnp.reshape(prelu_a.astype(jnp.float32), (1, 1))
    e1p = jnp.pad(emb1, ((0, 8 - emb1.shape[0]), (0, 0)))
    e2p = jnp.pad(emb2, ((0, 8 - emb2.shape[0]), (0, 0)))
    zrows = jnp.zeros((ROWS_PER_TILE, D), jnp.float32)
    zflat = jnp.zeros((CNT_PER_TILE,), jnp.float32)
    ones128 = jnp.ones((C,), jnp.float32)

    h = pl.pallas_call(
        _enc_body,
        grid=(N // RB,),
        in_specs=[
            pl.BlockSpec((RB, D), lambda i: (i, 0)),
            pl.BlockSpec((D, D), lambda i: (0, 0)),
            pl.BlockSpec((MP // 128, 128), lambda i: (0, 0)),
            pl.BlockSpec((1, 1), lambda i: (0, 0)),
        ],
        out_specs=pl.BlockSpec((RB, D), lambda i: (i, 0)),
        out_shape=jax.ShapeDtypeStruct((N, D), jnp.float32),
    )(x, W_enc, mi, a11)

    aggr, c1f, c2f = _make_sc_aggregate(CH0, CH1)(epk, h, zrows, zflat, ones128)
    c1 = c1f.reshape(NC, NP, 8)
    c2 = c2f.reshape(NC, NP, 8)

    out = pl.pallas_call(
        _mlp_body,
        grid=(N // RB,),
        in_specs=[
            pl.BlockSpec((NC, RB, D), lambda i: (0, i, 0)),
            pl.BlockSpec((NC, RB, 8), lambda i: (0, i, 0)),
            pl.BlockSpec((NC, RB, 8), lambda i: (0, i, 0)),
            pl.BlockSpec((RB, D), lambda i: (i, 0)),
            pl.BlockSpec((8, D), lambda i: (0, 0)),
            pl.BlockSpec((8, D), lambda i: (0, 0)),
            pl.BlockSpec((2 * D, D), lambda i: (0, 0)),
            pl.BlockSpec((1, 2 * D), lambda i: (0, 0)),
            pl.BlockSpec((D, 2 * D), lambda i: (0, 0)),
            pl.BlockSpec((1, D), lambda i: (0, 0)),
        ],
        out_specs=pl.BlockSpec((RB, D), lambda i: (i, 0)),
        out_shape=jax.ShapeDtypeStruct((N, D), jnp.float32),
    )(aggr, c1, c2, h, e1p, e2p, W1, b1.reshape(1, 2 * D), W2,
      b2.reshape(1, D))
    return out
